# baseline (speedup 1.0000x reference)
import functools

import jax
import jax.numpy as jnp
from jax import lax
from jax.experimental import pallas as pl
from jax.experimental.pallas import tpu as pltpu
from jax.experimental.pallas import tpu_sc as plsc

_LANES = 16


def _sc_gather(t, table_a, table_c):
    B = t.shape[0]
    info = plsc.get_sparse_core_info()
    nc, ns = info.num_cores, info.num_subcores
    nw = nc * ns
    b_per_w = B // nw

    mesh = plsc.VectorSubcoreMesh(core_axis_name="c", subcore_axis_name="s")

    @functools.partial(
        pl.kernel,
        mesh=mesh,
        out_type=(
            jax.ShapeDtypeStruct((B,), jnp.float32),
            jax.ShapeDtypeStruct((B,), jnp.float32),
        ),
        scratch_types=[
            pltpu.VMEM((b_per_w,), jnp.int32),
            pltpu.VMEM((b_per_w,), jnp.float32),
            pltpu.VMEM((b_per_w,), jnp.float32),
            pltpu.SemaphoreType.DMA,
        ],
    )
    def gather_kernel(t_hbm, a_hbm, c_hbm, oa_hbm, oc_hbm,
                      idx_v, oa_v, oc_v, sem):
        wid = lax.axis_index("s") * nc + lax.axis_index("c")
        base = wid * b_per_w
        pltpu.sync_copy(t_hbm.at[pl.ds(base, b_per_w)], idx_v)
        cp_a = pltpu.async_copy(a_hbm.at[idx_v], oa_v, sem)
        cp_c = pltpu.async_copy(c_hbm.at[idx_v], oc_v, sem)
        cp_a.wait()
        cp_c.wait()
        pltpu.sync_copy(oa_v, oa_hbm.at[pl.ds(base, b_per_w)])
        pltpu.sync_copy(oc_v, oc_hbm.at[pl.ds(base, b_per_w)])

    return gather_kernel(t, table_a, table_c)


def _blend_body(a_ref, c_ref, x_ref, n_ref, o_ref, no_ref):
    nv = n_ref[...]
    o_ref[...] = a_ref[...] * x_ref[...] + c_ref[...] * nv
    no_ref[...] = nv


def _tc_blend(xt, nt, a3, c3, lblk):
    L, D, B = xt.shape
    return pl.pallas_call(
        _blend_body,
        grid=(L // lblk,),
        in_specs=[
            pl.BlockSpec((1, 1, B), lambda i: (0, 0, 0)),
            pl.BlockSpec((1, 1, B), lambda i: (0, 0, 0)),
            pl.BlockSpec((lblk, D, B), lambda i: (i, 0, 0)),
            pl.BlockSpec((lblk, D, B), lambda i: (i, 0, 0)),
        ],
        out_specs=[
            pl.BlockSpec((lblk, D, B), lambda i: (i, 0, 0)),
            pl.BlockSpec((lblk, D, B), lambda i: (i, 0, 0)),
        ],
        out_shape=[
            jax.ShapeDtypeStruct((L, D, B), jnp.float32),
            jax.ShapeDtypeStruct((L, D, B), jnp.float32),
        ],
    )(a3, c3, xt, nt)


def kernel(x_0, t, noise, sqrt_alphas_cumprod, sqrt_one_minus_alphas_cumprod):
    B, L, D = x_0.shape
    t32 = t.astype(jnp.int32)
    a_g, c_g = _sc_gather(t32, sqrt_alphas_cumprod,
                          sqrt_one_minus_alphas_cumprod)
    xt = jnp.transpose(x_0, (1, 2, 0))
    nt = jnp.transpose(noise, (1, 2, 0))
    out_t, nout_t = _tc_blend(xt, nt, a_g.reshape(1, 1, B),
                              c_g.reshape(1, 1, B), 2)
    return jnp.transpose(out_t, (2, 0, 1)), jnp.transpose(nout_t, (2, 0, 1))


# speedup vs baseline: 1.0136x; 1.0136x over previous
import functools

import jax
import jax.numpy as jnp
from jax import lax
from jax.experimental import pallas as pl
from jax.experimental.pallas import tpu as pltpu
from jax.experimental.pallas import tpu_sc as plsc

_LANES = 16


def _sc_gather(t, table_a, table_c):
    B = t.shape[0]
    info = plsc.get_sparse_core_info()
    nc, ns = info.num_cores, info.num_subcores
    nw = nc * ns
    b_per_w = B // nw

    mesh = plsc.VectorSubcoreMesh(core_axis_name="c", subcore_axis_name="s")

    @functools.partial(
        pl.kernel,
        mesh=mesh,
        out_type=(
            jax.ShapeDtypeStruct((B,), jnp.float32),
            jax.ShapeDtypeStruct((B,), jnp.float32),
        ),
        scratch_types=[
            pltpu.VMEM((b_per_w,), jnp.int32),
            pltpu.VMEM((b_per_w,), jnp.float32),
            pltpu.VMEM((b_per_w,), jnp.float32),
            pltpu.SemaphoreType.DMA,
        ],
    )
    def gather_kernel(t_hbm, a_hbm, c_hbm, oa_hbm, oc_hbm,
                      idx_v, oa_v, oc_v, sem):
        wid = lax.axis_index("s") * nc + lax.axis_index("c")
        base = wid * b_per_w
        pltpu.sync_copy(t_hbm.at[pl.ds(base, b_per_w)], idx_v)
        cp_a = pltpu.async_copy(a_hbm.at[idx_v], oa_v, sem)
        cp_c = pltpu.async_copy(c_hbm.at[idx_v], oc_v, sem)
        cp_a.wait()
        cp_c.wait()
        pltpu.sync_copy(oa_v, oa_hbm.at[pl.ds(base, b_per_w)])
        pltpu.sync_copy(oc_v, oc_hbm.at[pl.ds(base, b_per_w)])

    return gather_kernel(t, table_a, table_c)


def _blend_body(a_ref, c_ref, x_ref, n_ref, o_ref, no_ref):
    nv = n_ref[...]
    o_ref[...] = a_ref[...] * x_ref[...] + c_ref[...] * nv
    no_ref[...] = nv


def _tc_blend(xt, nt, a3, c3, lblk):
    L, D, B = xt.shape
    return pl.pallas_call(
        _blend_body,
        grid=(L // lblk,),
        in_specs=[
            pl.BlockSpec((1, 1, B), lambda i: (0, 0, 0)),
            pl.BlockSpec((1, 1, B), lambda i: (0, 0, 0)),
            pl.BlockSpec((lblk, D, B), lambda i: (i, 0, 0)),
            pl.BlockSpec((lblk, D, B), lambda i: (i, 0, 0)),
        ],
        out_specs=[
            pl.BlockSpec((lblk, D, B), lambda i: (i, 0, 0)),
            pl.BlockSpec((lblk, D, B), lambda i: (i, 0, 0)),
        ],
        out_shape=[
            jax.ShapeDtypeStruct((L, D, B), jnp.float32),
            jax.ShapeDtypeStruct((L, D, B), jnp.float32),
        ],
    )(a3, c3, xt, nt)


def kernel(x_0, t, noise, sqrt_alphas_cumprod, sqrt_one_minus_alphas_cumprod):
    B, L, D = x_0.shape
    t32 = t.astype(jnp.int32)
    a_g, c_g = _sc_gather(t32, sqrt_alphas_cumprod,
                          sqrt_one_minus_alphas_cumprod)
    xt = jnp.transpose(x_0, (1, 2, 0))
    nt = jnp.transpose(noise, (1, 2, 0))
    out_t, nout_t = _tc_blend(xt, nt, a_g.reshape(1, 1, B),
                              c_g.reshape(1, 1, B), 5)
    return jnp.transpose(out_t, (2, 0, 1)), jnp.transpose(nout_t, (2, 0, 1))


# speedup vs baseline: 1.0206x; 1.0069x over previous
import functools

import jax
import jax.numpy as jnp
from jax import lax
from jax.experimental import pallas as pl
from jax.experimental.pallas import tpu as pltpu
from jax.experimental.pallas import tpu_sc as plsc

_LANES = 16


def _sc_gather(t, table_a, table_c):
    B = t.shape[0]
    info = plsc.get_sparse_core_info()
    nc, ns = info.num_cores, info.num_subcores
    nw = nc * ns
    b_per_w = B // nw

    mesh = plsc.VectorSubcoreMesh(core_axis_name="c", subcore_axis_name="s")

    @functools.partial(
        pl.kernel,
        mesh=mesh,
        out_type=(
            jax.ShapeDtypeStruct((B,), jnp.float32),
            jax.ShapeDtypeStruct((B,), jnp.float32),
        ),
        scratch_types=[
            pltpu.VMEM((b_per_w,), jnp.int32),
            pltpu.VMEM((b_per_w,), jnp.float32),
            pltpu.VMEM((b_per_w,), jnp.float32),
            pltpu.SemaphoreType.DMA,
            pltpu.SemaphoreType.DMA,
            pltpu.SemaphoreType.DMA,
        ],
    )
    def gather_kernel(t_hbm, a_hbm, c_hbm, oa_hbm, oc_hbm,
                      idx_v, oa_v, oc_v, sem_a, sem_c, sem_w):
        wid = lax.axis_index("s") * nc + lax.axis_index("c")
        base = wid * b_per_w
        pltpu.sync_copy(t_hbm.at[pl.ds(base, b_per_w)], idx_v)
        cp_a = pltpu.async_copy(a_hbm.at[idx_v], oa_v, sem_a)
        cp_c = pltpu.async_copy(c_hbm.at[idx_v], oc_v, sem_c)
        cp_a.wait()
        wr_a = pltpu.async_copy(oa_v, oa_hbm.at[pl.ds(base, b_per_w)], sem_w)
        cp_c.wait()
        wr_c = pltpu.async_copy(oc_v, oc_hbm.at[pl.ds(base, b_per_w)], sem_w)
        wr_a.wait()
        wr_c.wait()

    return gather_kernel(t, table_a, table_c)


def _tc_blend(xt, nt, a3, c3, lblk, in_bufs=3):
    L, D, B = xt.shape

    def outer(a_v, c_v, x_hbm, n_hbm, o_hbm, no_hbm):
        def inner(x_ref, n_ref, o_ref, no_ref):
            nv = n_ref[...]
            o_ref[...] = a_v[...] * x_ref[...] + c_v[...] * nv
            no_ref[...] = nv

        pipe = pltpu.emit_pipeline(
            inner,
            grid=(L // lblk,),
            in_specs=[
                pl.BlockSpec((lblk, D, B), lambda i: (i, 0, 0),
                             pipeline_mode=pl.Buffered(buffer_count=in_bufs)),
                pl.BlockSpec((lblk, D, B), lambda i: (i, 0, 0),
                             pipeline_mode=pl.Buffered(buffer_count=in_bufs)),
            ],
            out_specs=[
                pl.BlockSpec((lblk, D, B), lambda i: (i, 0, 0)),
                pl.BlockSpec((lblk, D, B), lambda i: (i, 0, 0)),
            ],
        )
        pipe(x_hbm, n_hbm, o_hbm, no_hbm)

    return pl.pallas_call(
        outer,
        in_specs=[
            pl.BlockSpec(memory_space=pltpu.VMEM),
            pl.BlockSpec(memory_space=pltpu.VMEM),
            pl.BlockSpec(memory_space=pl.ANY),
            pl.BlockSpec(memory_space=pl.ANY),
        ],
        out_specs=[
            pl.BlockSpec(memory_space=pl.ANY),
            pl.BlockSpec(memory_space=pl.ANY),
        ],
        out_shape=[
            jax.ShapeDtypeStruct((L, D, B), jnp.float32),
            jax.ShapeDtypeStruct((L, D, B), jnp.float32),
        ],
    )(a3, c3, xt, nt)


def kernel(x_0, t, noise, sqrt_alphas_cumprod, sqrt_one_minus_alphas_cumprod):
    B, L, D = x_0.shape
    t32 = t.astype(jnp.int32)
    a_g, c_g = _sc_gather(t32, sqrt_alphas_cumprod,
                          sqrt_one_minus_alphas_cumprod)
    xt = jnp.transpose(x_0, (1, 2, 0))
    nt = jnp.transpose(noise, (1, 2, 0))
    out_t, nout_t = _tc_blend(xt, nt, a_g.reshape(1, 1, B),
                              c_g.reshape(1, 1, B), 5)
    return jnp.transpose(out_t, (2, 0, 1)), jnp.transpose(nout_t, (2, 0, 1))
